# baseline (speedup 1.0000x reference)
import functools

import jax
import jax.numpy as jnp
from jax import lax
from jax.experimental import pallas as pl
from jax.experimental.pallas import tpu as pltpu
from jax.experimental.pallas import tpu_sc as plsc

N = 10000
E = 320000
D = 128
H = 64
A = 5

NC = 2
NS = 16
NW = NC * NS
ER = E // 128
CH = 80
NH = CH // 2
REM = ER - (NW - 1) * CH
NP = 10240
RPT = NP // NS


def _sc_aggregate(x, ei_r):
    mesh = plsc.VectorSubcoreMesh(core_axis_name="c", subcore_axis_name="s")

    @functools.partial(
        pl.kernel,
        mesh=mesh,
        out_type=[
            jax.ShapeDtypeStruct((NC, NP, D), jnp.float32),
            jax.ShapeDtypeStruct((NC, NP), jnp.float32),
        ],
        scratch_types=[
            pltpu.VMEM((NH, 128), jnp.int32),
            pltpu.VMEM((NH, 128), jnp.int32),
            pltpu.VMEM((2, 128, D), jnp.float32),
            pltpu.VMEM((RPT,), jnp.float32),
            pltpu.VMEM((128,), jnp.float32),
            pltpu.VMEM_SHARED((NP, D), jnp.float32),
            pltpu.VMEM_SHARED((NP,), jnp.float32),
            pltpu.SemaphoreType.DMA,
            pltpu.SemaphoreType.DMA,
            pltpu.SemaphoreType.DMA,
        ],
    )
    def k(x_hbm, ei_hbm, p_hbm, dg_hbm,
          src_v, dst_v, rows_v, zdeg_v, ones_v, agg_sh, deg_sh,
          gsem, ssem, dsem):
        c = lax.axis_index("c")
        s = lax.axis_index("s")
        wid = c * NS + s

        z16 = jnp.zeros((16,), jnp.float32)
        o16 = jnp.ones((16,), jnp.float32)

        def zrow_body(i, _):
            rows_v[0, i // 8, pl.ds((i % 8) * 16, 16)] = z16
            return 0
        lax.fori_loop(0, 128 * 8, zrow_body, 0)

        def zdeg_body(i, _):
            zdeg_v[pl.ds(i * 16, 16)] = z16
            return 0
        lax.fori_loop(0, RPT // 16, zdeg_body, 0)

        def ones_body(i, _):
            ones_v[pl.ds(i * 16, 16)] = o16
            return 0
        lax.fori_loop(0, 8, ones_body, 0)

        base = s * RPT
        for j in range(RPT // 128):
            pltpu.sync_copy(rows_v.at[0], agg_sh.at[pl.ds(base + j * 128, 128)])
        pltpu.sync_copy(zdeg_v, deg_sh.at[pl.ds(base, RPT)])
        plsc.subcore_barrier()

        def run_slab(row0, n):
            pltpu.sync_copy(ei_hbm.at[0, pl.ds(row0, n)],
                            src_v.at[pl.ds(0, n)])
            pltpu.sync_copy(ei_hbm.at[1, pl.ds(row0, n)],
                            dst_v.at[pl.ds(0, n)])

            pltpu.async_copy(x_hbm.at[src_v.at[0]], rows_v.at[0], gsem)

            def edge_body(j, _):
                b = lax.rem(j, 2)

                @pl.when(j >= 1)
                def _():
                    pltpu.make_async_copy(
                        rows_v.at[1 - b], agg_sh.at[dst_v.at[0]], ssem).wait()

                @pl.when(j <= n - 2)
                def _():
                    pltpu.async_copy(
                        x_hbm.at[src_v.at[j + 1]], rows_v.at[1 - b], gsem)

                pltpu.make_async_copy(
                    x_hbm.at[src_v.at[j]], rows_v.at[b], gsem).wait()
                pltpu.async_copy(
                    rows_v.at[b], agg_sh.at[dst_v.at[j]], ssem, add=True)
                pltpu.async_copy(
                    ones_v, deg_sh.at[dst_v.at[j]], dsem, add=True)
                return 0
            lax.fori_loop(0, n, edge_body, 0)

            pltpu.make_async_copy(
                rows_v.at[(n - 1) % 2], agg_sh.at[dst_v.at[0]], ssem).wait()

            def ddrain(j, _):
                pltpu.make_async_copy(
                    ones_v, deg_sh.at[dst_v.at[0]], dsem).wait()
                return 0
            lax.fori_loop(0, n, ddrain, 0)

        @pl.when(wid < NW - 1)
        def _():
            for half in range(2):
                run_slab(wid * CH + half * NH, NH)

        @pl.when(wid == NW - 1)
        def _():
            run_slab((NW - 1) * CH, REM)

        plsc.subcore_barrier()

        pltpu.sync_copy(agg_sh.at[pl.ds(base, RPT)],
                        p_hbm.at[c, pl.ds(base, RPT)])
        pltpu.sync_copy(deg_sh.at[pl.ds(base, RPT)],
                        dg_hbm.at[c, pl.ds(base, RPT)])

    return k(x, ei_r)


def _tc_head(p, dgt, w_enc, b_enc, w1p, b1p, w2p, b2p):
    B = 1000
    grid = N // B

    def dot3(a, w):
        ah = a.astype(jnp.bfloat16)
        al = (a - ah.astype(jnp.float32)).astype(jnp.bfloat16)
        wh = w.astype(jnp.bfloat16)
        wl = (w - wh.astype(jnp.float32)).astype(jnp.bfloat16)
        f32 = jnp.float32
        return (jnp.dot(ah, wh, preferred_element_type=f32)
                + jnp.dot(ah, wl, preferred_element_type=f32)
                + jnp.dot(al, wh, preferred_element_type=f32))

    def body(p_ref, dgt_ref, we_ref, be_ref, w1_ref, b1_ref, w2_ref, b2_ref,
             o_ref):
        deg = dgt_ref[:, 0:1] + dgt_ref[:, 1:2]
        agg = (p_ref[0] + p_ref[1]) / jnp.maximum(deg, 1.0)
        h = jnp.maximum(dot3(agg, we_ref[...]) + be_ref[...], 0.0)
        z = jnp.maximum(dot3(h, w1_ref[...]) + b1_ref[...], 0.0)
        o_ref[...] = (dot3(z, w2_ref[...]) + b2_ref[...])[:, :A]

    return pl.pallas_call(
        body,
        grid=(grid,),
        in_specs=[
            pl.BlockSpec((NC, B, D), lambda i: (0, i, 0)),
            pl.BlockSpec((B, NC), lambda i: (i, 0)),
            pl.BlockSpec((D, D), lambda i: (0, 0)),
            pl.BlockSpec((1, D), lambda i: (0, 0)),
            pl.BlockSpec((D, H), lambda i: (0, 0)),
            pl.BlockSpec((1, H), lambda i: (0, 0)),
            pl.BlockSpec((H, 8), lambda i: (0, 0)),
            pl.BlockSpec((1, 8), lambda i: (0, 0)),
        ],
        out_specs=pl.BlockSpec((B, A), lambda i: (i, 0)),
        out_shape=jax.ShapeDtypeStruct((N, A), jnp.float32),
    )(p, dgt, w_enc, b_enc, w1p, b1p, w2p, b2p)


def kernel(x, edge_index, W_enc, b_enc, W1, b1, W2, b2):
    p, dg = _sc_aggregate(x, edge_index.reshape(2, ER, 128))

    b1p = b1.reshape(1, H)
    w2p = jnp.pad(W2, ((0, 0), (0, 8 - A)))
    b2p = jnp.pad(b2, (0, 8 - A)).reshape(1, 8)

    return _tc_head(p, dg.T, W_enc, b_enc.reshape(1, D), W1, b1p, w2p, b2p)


# speedup vs baseline: 1.0193x; 1.0193x over previous
import functools

import jax
import jax.numpy as jnp
from jax import lax
from jax.experimental import pallas as pl
from jax.experimental.pallas import tpu as pltpu
from jax.experimental.pallas import tpu_sc as plsc

N = 10000
E = 320000
D = 128
H = 64
A = 5

NC = 2
NS = 16
NW = NC * NS
ER = E // 128
CH = 80
NH = CH // 2
REM = ER - (NW - 1) * CH
NP = 10240
RPT = NP // NS


def _sc_aggregate(x, ei_r):
    mesh = plsc.VectorSubcoreMesh(core_axis_name="c", subcore_axis_name="s")

    @functools.partial(
        pl.kernel,
        mesh=mesh,
        out_type=[
            jax.ShapeDtypeStruct((NC, NP, D), jnp.float32),
            jax.ShapeDtypeStruct((NC, NP), jnp.float32),
        ],
        scratch_types=[
            pltpu.VMEM((NH, 128), jnp.int32),
            pltpu.VMEM((NH, 128), jnp.int32),
            pltpu.VMEM((2, 128, D), jnp.float32),
            pltpu.VMEM((RPT,), jnp.float32),
            pltpu.VMEM((128,), jnp.float32),
            pltpu.VMEM_SHARED((NP, D), jnp.float32),
            pltpu.VMEM_SHARED((NP,), jnp.float32),
            pltpu.SemaphoreType.DMA,
            pltpu.SemaphoreType.DMA,
            pltpu.SemaphoreType.DMA,
        ],
    )
    def k(x_hbm, ei_hbm, p_hbm, dg_hbm,
          src_v, dst_v, rows_v, zdeg_v, ones_v, agg_sh, deg_sh,
          gsem, ssem, dsem):
        c = lax.axis_index("c")
        s = lax.axis_index("s")
        wid = c * NS + s

        z16 = jnp.zeros((16,), jnp.float32)
        o16 = jnp.ones((16,), jnp.float32)

        def zrow_body(i, _):
            rows_v[0, i // 8, pl.ds((i % 8) * 16, 16)] = z16
            return 0
        lax.fori_loop(0, 128 * 8, zrow_body, 0)

        def zdeg_body(i, _):
            zdeg_v[pl.ds(i * 16, 16)] = z16
            return 0
        lax.fori_loop(0, RPT // 16, zdeg_body, 0)

        def ones_body(i, _):
            ones_v[pl.ds(i * 16, 16)] = o16
            return 0
        lax.fori_loop(0, 8, ones_body, 0)

        base = s * RPT
        for j in range(RPT // 128):
            pltpu.sync_copy(rows_v.at[0], agg_sh.at[pl.ds(base + j * 128, 128)])
        pltpu.sync_copy(zdeg_v, deg_sh.at[pl.ds(base, RPT)])
        plsc.subcore_barrier()

        def run_slab(row0, n):
            def idx_ld(r, _):
                pltpu.async_copy(
                    ei_hbm.at[0, pl.ds((row0 + r) * 128, 128)],
                    src_v.at[r], gsem)
                pltpu.async_copy(
                    ei_hbm.at[1, pl.ds((row0 + r) * 128, 128)],
                    dst_v.at[r], gsem)
                return 0
            lax.fori_loop(0, n, idx_ld, 0)

            def idx_drain(r, _):
                pltpu.make_async_copy(
                    ei_hbm.at[0, pl.ds(row0 * 128, 128)],
                    src_v.at[0], gsem).wait()
                pltpu.make_async_copy(
                    ei_hbm.at[1, pl.ds(row0 * 128, 128)],
                    dst_v.at[0], gsem).wait()
                return 0
            lax.fori_loop(0, n, idx_drain, 0)

            pltpu.async_copy(x_hbm.at[src_v.at[0]], rows_v.at[0], gsem)

            def edge_body(j, _):
                b = lax.rem(j, 2)

                @pl.when(j >= 1)
                def _():
                    pltpu.make_async_copy(
                        rows_v.at[1 - b], agg_sh.at[dst_v.at[0]], ssem).wait()

                @pl.when(j <= n - 2)
                def _():
                    pltpu.async_copy(
                        x_hbm.at[src_v.at[j + 1]], rows_v.at[1 - b], gsem)

                pltpu.make_async_copy(
                    x_hbm.at[src_v.at[j]], rows_v.at[b], gsem).wait()
                pltpu.async_copy(
                    rows_v.at[b], agg_sh.at[dst_v.at[j]], ssem, add=True)
                pltpu.async_copy(
                    ones_v, deg_sh.at[dst_v.at[j]], dsem, add=True)
                return 0
            lax.fori_loop(0, n, edge_body, 0)

            pltpu.make_async_copy(
                rows_v.at[(n - 1) % 2], agg_sh.at[dst_v.at[0]], ssem).wait()

            def ddrain(j, _):
                pltpu.make_async_copy(
                    ones_v, deg_sh.at[dst_v.at[0]], dsem).wait()
                return 0
            lax.fori_loop(0, n, ddrain, 0)

        @pl.when(wid < NW - 1)
        def _():
            for half in range(2):
                run_slab(wid * CH + half * NH, NH)

        @pl.when(wid == NW - 1)
        def _():
            run_slab((NW - 1) * CH, REM)

        plsc.subcore_barrier()

        pltpu.sync_copy(agg_sh.at[pl.ds(base, RPT)],
                        p_hbm.at[c, pl.ds(base, RPT)])
        pltpu.sync_copy(deg_sh.at[pl.ds(base, RPT)],
                        dg_hbm.at[c, pl.ds(base, RPT)])

    return k(x, ei_r)


def _tc_head(p, dgt, w_enc, b_enc, w1p, b1p, w2p, b2p):
    B = 1000
    grid = N // B

    def dot3(a, w):
        ah = a.astype(jnp.bfloat16)
        al = (a - ah.astype(jnp.float32)).astype(jnp.bfloat16)
        wh = w.astype(jnp.bfloat16)
        wl = (w - wh.astype(jnp.float32)).astype(jnp.bfloat16)
        f32 = jnp.float32
        return (jnp.dot(ah, wh, preferred_element_type=f32)
                + jnp.dot(ah, wl, preferred_element_type=f32)
                + jnp.dot(al, wh, preferred_element_type=f32))

    def body(p_ref, dgt_ref, we_ref, be_ref, w1_ref, b1_ref, w2_ref, b2_ref,
             o_ref):
        deg = dgt_ref[:, 0:1] + dgt_ref[:, 1:2]
        agg = (p_ref[0] + p_ref[1]) / jnp.maximum(deg, 1.0)
        h = jnp.maximum(dot3(agg, we_ref[...]) + be_ref[...], 0.0)
        z = jnp.maximum(dot3(h, w1_ref[...]) + b1_ref[...], 0.0)
        o_ref[...] = (dot3(z, w2_ref[...]) + b2_ref[...])[:, :A]

    return pl.pallas_call(
        body,
        grid=(grid,),
        in_specs=[
            pl.BlockSpec((NC, B, D), lambda i: (0, i, 0)),
            pl.BlockSpec((B, NC), lambda i: (i, 0)),
            pl.BlockSpec((D, D), lambda i: (0, 0)),
            pl.BlockSpec((1, D), lambda i: (0, 0)),
            pl.BlockSpec((D, H), lambda i: (0, 0)),
            pl.BlockSpec((1, H), lambda i: (0, 0)),
            pl.BlockSpec((H, 8), lambda i: (0, 0)),
            pl.BlockSpec((1, 8), lambda i: (0, 0)),
        ],
        out_specs=pl.BlockSpec((B, A), lambda i: (i, 0)),
        out_shape=jax.ShapeDtypeStruct((N, A), jnp.float32),
    )(p, dgt, w_enc, b_enc, w1p, b1p, w2p, b2p)


def kernel(x, edge_index, W_enc, b_enc, W1, b1, W2, b2):
    p, dg = _sc_aggregate(x, edge_index)

    b1p = b1.reshape(1, H)
    w2p = jnp.pad(W2, ((0, 0), (0, 8 - A)))
    b2p = jnp.pad(b2, (0, 8 - A)).reshape(1, 8)

    return _tc_head(p, dg.T, W_enc, b_enc.reshape(1, D), W1, b1p, w2p, b2p)


# speedup vs baseline: 1.0840x; 1.0635x over previous
import functools

import jax
import jax.numpy as jnp
from jax import lax
from jax.experimental import pallas as pl
from jax.experimental.pallas import tpu as pltpu
from jax.experimental.pallas import tpu_sc as plsc

N = 10000
E = 320000
D = 128
H = 64
A = 5

NC = 2
NS = 16
NW = NC * NS
ER = E // 128
CH = 80
NH = CH // 2
REM = ER - (NW - 1) * CH
NP = 10240
RPT = NP // NS


def _sc_aggregate(x, ei_r):
    mesh = plsc.VectorSubcoreMesh(core_axis_name="c", subcore_axis_name="s")

    @functools.partial(
        pl.kernel,
        mesh=mesh,
        out_type=[
            jax.ShapeDtypeStruct((NC, NP, D), jnp.float32),
            jax.ShapeDtypeStruct((NC, NP), jnp.float32),
        ],
        scratch_types=[
            pltpu.VMEM((NH, 128), jnp.int32),
            pltpu.VMEM((NH, 128), jnp.int32),
            pltpu.VMEM((2, 128, D), jnp.float32),
            pltpu.VMEM((RPT,), jnp.float32),
            pltpu.VMEM((128,), jnp.float32),
            pltpu.VMEM_SHARED((NP, D), jnp.float32),
            pltpu.VMEM_SHARED((NP,), jnp.float32),
            pltpu.SemaphoreType.DMA,
            pltpu.SemaphoreType.DMA,
            pltpu.SemaphoreType.DMA,
        ],
    )
    def k(x_hbm, ei_hbm, p_hbm, dg_hbm,
          src_v, dst_v, rows_v, zdeg_v, ones_v, agg_sh, deg_sh,
          gsem, ssem, dsem):
        c = lax.axis_index("c")
        s = lax.axis_index("s")
        wid = c * NS + s

        z16 = jnp.zeros((16,), jnp.float32)
        o16 = jnp.ones((16,), jnp.float32)

        def zrow_body(i, _):
            rows_v[0, i // 8, pl.ds((i % 8) * 16, 16)] = z16
            return 0
        lax.fori_loop(0, 128 * 8, zrow_body, 0)

        def zdeg_body(i, _):
            zdeg_v[pl.ds(i * 16, 16)] = z16
            return 0
        lax.fori_loop(0, RPT // 16, zdeg_body, 0)

        def ones_body(i, _):
            ones_v[pl.ds(i * 16, 16)] = o16
            return 0
        lax.fori_loop(0, 8, ones_body, 0)

        base = s * RPT
        for j in range(RPT // 128):
            pltpu.sync_copy(rows_v.at[0], agg_sh.at[pl.ds(base + j * 128, 128)])
        pltpu.sync_copy(zdeg_v, deg_sh.at[pl.ds(base, RPT)])
        plsc.subcore_barrier()

        def run_slab(row0, n):
            def idx_ld(r, _):
                pltpu.async_copy(
                    ei_hbm.at[0, pl.ds((row0 + r) * 128, 128)],
                    src_v.at[r], gsem)
                pltpu.async_copy(
                    ei_hbm.at[1, pl.ds((row0 + r) * 128, 128)],
                    dst_v.at[r], gsem)
                return 0
            lax.fori_loop(0, n, idx_ld, 0)

            def idx_drain(r, _):
                pltpu.make_async_copy(
                    ei_hbm.at[0, pl.ds(row0 * 128, 128)],
                    src_v.at[0], gsem).wait()
                pltpu.make_async_copy(
                    ei_hbm.at[1, pl.ds(row0 * 128, 128)],
                    dst_v.at[0], gsem).wait()
                return 0
            lax.fori_loop(0, n, idx_drain, 0)

            pltpu.async_copy(x_hbm.at[src_v.at[0]], rows_v.at[0], gsem)

            def edge_body(j, _):
                b = lax.rem(j, 2)

                @pl.when(j >= 1)
                def _():
                    pltpu.make_async_copy(
                        rows_v.at[1 - b], agg_sh.at[dst_v.at[0]], ssem).wait()

                @pl.when(j <= n - 2)
                def _():
                    pltpu.async_copy(
                        x_hbm.at[src_v.at[j + 1]], rows_v.at[1 - b], gsem)

                pltpu.make_async_copy(
                    x_hbm.at[src_v.at[j]], rows_v.at[b], gsem).wait()
                pltpu.async_copy(
                    rows_v.at[b], agg_sh.at[dst_v.at[j]], ssem, add=True)
                pltpu.async_copy(
                    ones_v, deg_sh.at[dst_v.at[j]], dsem, add=True)
                return 0
            lax.fori_loop(0, n, edge_body, 0)

            pltpu.make_async_copy(
                rows_v.at[(n - 1) % 2], agg_sh.at[dst_v.at[0]], ssem).wait()

            def ddrain(j, _):
                pltpu.make_async_copy(
                    ones_v, deg_sh.at[dst_v.at[0]], dsem).wait()
                return 0
            lax.fori_loop(0, n, ddrain, 0)

        @pl.when(wid < NW - 1)
        def _():
            for half in range(2):
                run_slab(wid * CH + half * NH, NH)

        @pl.when(wid == NW - 1)
        def _():
            run_slab((NW - 1) * CH, REM)

        plsc.subcore_barrier()

        pltpu.sync_copy(agg_sh.at[pl.ds(base, RPT)],
                        p_hbm.at[c, pl.ds(base, RPT)])
        pltpu.sync_copy(deg_sh.at[pl.ds(base, RPT)],
                        dg_hbm.at[c, pl.ds(base, RPT)])

    return k(x, ei_r)


def _tc_head(p, dgt, w_enc, b_enc, w1p, b1p, w2p, b2p):
    B = 2048
    grid = (N + B - 1) // B

    def dot3(a, w):
        ah = a.astype(jnp.bfloat16)
        al = (a - ah.astype(jnp.float32)).astype(jnp.bfloat16)
        wh = w.astype(jnp.bfloat16)
        wl = (w - wh.astype(jnp.float32)).astype(jnp.bfloat16)
        f32 = jnp.float32
        return (jnp.dot(ah, wh, preferred_element_type=f32)
                + jnp.dot(ah, wl, preferred_element_type=f32)
                + jnp.dot(al, wh, preferred_element_type=f32))

    def body(p_ref, dgt_ref, we_ref, be_ref, w1_ref, b1_ref, w2_ref, b2_ref,
             o_ref):
        degr = dgt_ref[0:1, :] + dgt_ref[1:2, :]
        deg = jnp.transpose(degr, (1, 0))
        agg = (p_ref[0] + p_ref[1]) / jnp.maximum(deg, 1.0)
        h = jnp.maximum(dot3(agg, we_ref[...]) + be_ref[...], 0.0)
        z = jnp.maximum(dot3(h, w1_ref[...]) + b1_ref[...], 0.0)
        o_ref[...] = (dot3(z, w2_ref[...]) + b2_ref[...])[:, :A]

    return pl.pallas_call(
        body,
        grid=(grid,),
        in_specs=[
            pl.BlockSpec((NC, B, D), lambda i: (0, i, 0)),
            pl.BlockSpec((NC, B), lambda i: (0, i)),
            pl.BlockSpec((D, D), lambda i: (0, 0)),
            pl.BlockSpec((1, D), lambda i: (0, 0)),
            pl.BlockSpec((D, H), lambda i: (0, 0)),
            pl.BlockSpec((1, H), lambda i: (0, 0)),
            pl.BlockSpec((H, 8), lambda i: (0, 0)),
            pl.BlockSpec((1, 8), lambda i: (0, 0)),
        ],
        out_specs=pl.BlockSpec((B, A), lambda i: (i, 0)),
        out_shape=jax.ShapeDtypeStruct((N, A), jnp.float32),
    )(p, dgt, w_enc, b_enc, w1p, b1p, w2p, b2p)


def kernel(x, edge_index, W_enc, b_enc, W1, b1, W2, b2):
    p, dg = _sc_aggregate(x, edge_index)

    b1p = b1.reshape(1, H)
    w2p = jnp.pad(W2, ((0, 0), (0, 8 - A)))
    b2p = jnp.pad(b2, (0, 8 - A)).reshape(1, 8)

    return _tc_head(p, dg, W_enc, b_enc.reshape(1, D), W1, b1p, w2p, b2p)
